# baseline (speedup 1.0000x reference)
import functools

import jax
import jax.numpy as jnp
from jax import lax
from jax.experimental import pallas as pl
from jax.experimental.pallas import tpu as pltpu
from jax.experimental.pallas import tpu_sc as plsc

E_DIM = 64
NUM_FEATS = 26
H_DIM = 512
NUM_LABELS = 80
BATCH = 4096

NC, NS = 2, 16
NW = NC * NS
ROWS = BATCH * NUM_FEATS
R_PER_W = ROWS // NW
CHUNK = 128
NCH = R_PER_W // CHUNK


def _gather_body(x_hbm, table_hbm, out_hbm, idx_v, buf, sem0, sem1):
    wid = lax.axis_index("s") * NC + lax.axis_index("c")
    base_row = wid * R_PER_W
    pltpu.sync_copy(x_hbm.at[wid], idx_v)
    sems = (sem0, sem1)
    pltpu.async_copy(table_hbm.at[idx_v.at[0]], buf.at[0], sems[0])
    pltpu.async_copy(table_hbm.at[idx_v.at[1]], buf.at[1], sems[1])

    def step(i, _):
        g = 2 * i
        for b in range(2):
            c = g + b
            pltpu.make_async_copy(
                table_hbm.at[idx_v.at[c]], buf.at[b], sems[b]).wait()
            pltpu.sync_copy(
                buf.at[b], out_hbm.at[pl.ds(base_row + c * CHUNK, CHUNK)])
            pltpu.async_copy(
                table_hbm.at[idx_v.at[c + 2]], buf.at[b], sems[b])
        return 0

    lax.fori_loop(0, (NCH - 2) // 2, step, 0)
    for b in range(2):
        c = NCH - 2 + b
        pltpu.make_async_copy(
            table_hbm.at[idx_v.at[c]], buf.at[b], sems[b]).wait()
        pltpu.sync_copy(
            buf.at[b], out_hbm.at[pl.ds(base_row + c * CHUNK, CHUNK)])


_gather = pl.kernel(
    _gather_body,
    out_type=jax.ShapeDtypeStruct((ROWS, E_DIM), jnp.float32),
    mesh=plsc.VectorSubcoreMesh(
        core_axis_name="c", subcore_axis_name="s",
        num_cores=NC, num_subcores=NS),
    scratch_types=[
        pltpu.VMEM((NCH, CHUNK), jnp.int32),
        pltpu.VMEM((2, CHUNK, E_DIM), jnp.float32),
        pltpu.SemaphoreType.DMA,
        pltpu.SemaphoreType.DMA,
    ],
    compiler_params=pltpu.CompilerParams(use_tc_tiling_on_sc=False),
)


def _mlp_body(f_ref, w1_ref, b1_ref, w2_ref, b2_ref, o_ref):
    h = jnp.dot(f_ref[...], w1_ref[...], preferred_element_type=jnp.float32)
    h = h + b1_ref[...]
    h = h * h * h
    o_ref[...] = (
        jnp.dot(h, w2_ref[...], preferred_element_type=jnp.float32)
        + b2_ref[...])


_BB = 512

_mlp = pl.pallas_call(
    _mlp_body,
    grid=(BATCH // _BB,),
    in_specs=[
        pl.BlockSpec((_BB, NUM_FEATS * E_DIM), lambda i: (i, 0)),
        pl.BlockSpec((NUM_FEATS * E_DIM, H_DIM), lambda i: (0, 0)),
        pl.BlockSpec((1, H_DIM), lambda i: (0, 0)),
        pl.BlockSpec((H_DIM, NUM_LABELS), lambda i: (0, 0)),
        pl.BlockSpec((1, NUM_LABELS), lambda i: (0, 0)),
    ],
    out_specs=pl.BlockSpec((_BB, NUM_LABELS), lambda i: (i, 0)),
    out_shape=jax.ShapeDtypeStruct((BATCH, NUM_LABELS), jnp.float32),
)


def kernel(x, table, W1, b1, W2, b2):
    xi = x.astype(jnp.int32).reshape(NW, NCH, CHUNK)
    flat_rows = _gather(xi, table)
    flat = flat_rows.reshape(BATCH, NUM_FEATS * E_DIM)
    return _mlp(flat, W1, b1.reshape(1, H_DIM), W2, b2.reshape(1, NUM_LABELS))
